# baseline (speedup 1.0000x reference)
import jax
import jax.numpy as jnp
from jax import lax
from jax.experimental import pallas as pl
from jax.experimental.pallas import tpu as pltpu
from jax.experimental.pallas import tpu_sc as plsc

_BATCH = 16384
_DIM = 128
_LANES = 16
_NUM_WORKERS = 32
_CHUNK = _BATCH // _NUM_WORKERS


def _sc_body(x_hbm, y_hbm, tab_hbm, out_hbm, xv, yv, tabv, outv):
    nc = 2
    wid = lax.axis_index("s") * nc + lax.axis_index("c")
    base = wid * _CHUNK
    pltpu.sync_copy(tab_hbm, tabv)
    pltpu.sync_copy(x_hbm.at[pl.ds(base, _CHUNK)], xv)
    pltpu.sync_copy(y_hbm.at[pl.ds(base, _CHUNK)], yv)

    acc = jnp.zeros((_LANES,), jnp.float32)
    for j in range(_DIM // _LANES):
        t = tabv[pl.ds(j * _LANES, _LANES)]
        acc = acc + t * t
    s = jnp.float32(0.0)
    for k in range(_LANES):
        s = s + acc[k]

    for i in range(_CHUNK // _LANES):
        xi = xv[pl.ds(i * _LANES, _LANES)]
        yi = yv[pl.ds(i * _LANES, _LANES)]
        xc = jnp.clip(xi, 0, 0)
        yc = jnp.clip(yi, 0, 0)
        m = jnp.logical_and(xc == 0, yc == 0)
        sv = jnp.full((_LANES,), s, jnp.float32)
        dots = jnp.where(m, sv, jnp.zeros((_LANES,), jnp.float32))
        outv[pl.ds(i * _LANES, _LANES)] = 1.0 / (1.0 + jnp.exp(-dots))

    pltpu.sync_copy(outv, out_hbm.at[pl.ds(base, _CHUNK)])


def kernel(batch_data, table):
    x = batch_data[0]
    y = batch_data[1]
    tab = table.reshape(_DIM)
    mesh = plsc.VectorSubcoreMesh(core_axis_name="c", subcore_axis_name="s")
    run = pl.kernel(
        _sc_body,
        mesh=mesh,
        out_type=jax.ShapeDtypeStruct((_BATCH,), jnp.float32),
        scratch_types=[
            pltpu.VMEM((_CHUNK,), jnp.int32),
            pltpu.VMEM((_CHUNK,), jnp.int32),
            pltpu.VMEM((_DIM,), jnp.float32),
            pltpu.VMEM((_CHUNK,), jnp.float32),
        ],
    )
    return run(x, y, tab)


# speedup vs baseline: 6.7677x; 6.7677x over previous
import jax
import jax.numpy as jnp
from jax.experimental import pallas as pl

_BATCH = 16384
_ROWS = 128
_COLS = 128


def _item2vec_kernel(bd_ref, tab_ref, out_ref):
    t = tab_ref[...]
    n = t.shape[0]
    xc = jnp.clip(bd_ref[0], 0, n - 1)
    yc = jnp.clip(bd_ref[1], 0, n - 1)
    dots = jnp.zeros(out_ref.shape, jnp.float32)
    for r in range(n):
        for q in range(n):
            g = jnp.sum(t[r, :] * t[q, :])
            m = jnp.logical_and(xc == r, yc == q)
            dots = dots + jnp.where(m, g, 0.0)
    out_ref[...] = jax.nn.sigmoid(dots)


def kernel(batch_data, table):
    bd = batch_data.reshape(2, _ROWS, _COLS)
    out = pl.pallas_call(
        _item2vec_kernel,
        out_shape=jax.ShapeDtypeStruct((_ROWS, _COLS), jnp.float32),
    )(bd, table)
    return out.reshape(_BATCH)


# speedup vs baseline: 14.9261x; 2.2055x over previous
import jax
import jax.numpy as jnp
from jax.experimental import pallas as pl

_BATCH = 16384
_ROWS = 128
_COLS = 128


def _item2vec_kernel(tab_ref, out_ref):
    t = tab_ref[...]
    g = jnp.sum(t * t)
    out_ref[...] = jnp.full(out_ref.shape, jax.nn.sigmoid(g), jnp.float32)


def kernel(batch_data, table):
    del batch_data
    out = pl.pallas_call(
        _item2vec_kernel,
        out_shape=jax.ShapeDtypeStruct((_ROWS, _COLS), jnp.float32),
    )(table)
    return out.reshape(_BATCH)
